# baseline (speedup 1.0000x reference)
import functools

import jax
import jax.numpy as jnp
from jax import lax
from jax.experimental import pallas as pl
from jax.experimental.pallas import tpu as pltpu
from jax.experimental.pallas import tpu_sc as plsc

_T, _H, _I, _E, _R, _K = 2048, 768, 1536, 8, 16, 2
_TM = 256
_NT = _T * _K // _TM + _E - 1
_NP = _NT * _TM
_NW = 32
_RPW = _T * _K // _NW


def _dotT(a, b):
    return jax.lax.dot_general(
        a, b, (((1,), (1,)), ((), ())), preferred_element_type=jnp.float32)


_TPW = _T // _NW


def _sc_sort_body(x_hbm, pose_hbm, poso_hbm, xs_hbm, pe_v, po_v, buf_v,
                  sem_e, sem_o):
    wid = lax.axis_index("s") * 2 + lax.axis_index("c")
    base = wid * _TPW
    pltpu.sync_copy(pose_hbm.at[pl.ds(base, _TPW)], pe_v)
    pltpu.sync_copy(poso_hbm.at[pl.ds(base, _TPW)], po_v)
    pltpu.sync_copy(x_hbm.at[pl.ds(base, _TPW)], buf_v)
    ce = pltpu.make_async_copy(buf_v, xs_hbm.at[pe_v], sem_e)
    co = pltpu.make_async_copy(buf_v, xs_hbm.at[po_v], sem_o)
    ce.start()
    co.start()
    ce.wait()
    co.wait()


def _sc_sort():
    return pl.kernel(
        _sc_sort_body,
        out_type=jax.ShapeDtypeStruct((_NP, _H // 2), jnp.int32),
        mesh=plsc.VectorSubcoreMesh(core_axis_name="c", subcore_axis_name="s"),
        scratch_types=[
            pltpu.VMEM((_TPW,), jnp.int32),
            pltpu.VMEM((_TPW,), jnp.int32),
            pltpu.VMEM((_TPW, _H // 2), jnp.int32),
            pltpu.SemaphoreType.DMA,
            pltpu.SemaphoreType.DMA,
        ],
    )


def _sc_unsort_body(rows_hbm, pos_hbm, flat_hbm, pos_v, buf_v, sem_g):
    wid = lax.axis_index("s") * 2 + lax.axis_index("c")
    base = wid * _RPW
    pltpu.sync_copy(pos_hbm.at[pl.ds(base, _RPW)], pos_v)
    pltpu.async_copy(rows_hbm.at[pos_v], buf_v, sem_g).wait()
    pltpu.sync_copy(buf_v, flat_hbm.at[pl.ds(base, _RPW)])


def _sc_unsort():
    return pl.kernel(
        _sc_unsort_body,
        out_type=jax.ShapeDtypeStruct((_T * _K, _H), jnp.float32),
        mesh=plsc.VectorSubcoreMesh(core_axis_name="c", subcore_axis_name="s"),
        scratch_types=[
            pltpu.VMEM((_RPW,), jnp.int32),
            pltpu.VMEM((_RPW, _H), jnp.float32),
            pltpu.SemaphoreType.DMA,
        ],
    )


def _start_fetch(wgu_hbm, wd_hbm, wgu_buf, wd_buf, sems, e, s):
    pltpu.make_async_copy(wgu_hbm.at[e], wgu_buf.at[s], sems.at[s]).start()
    pltpu.make_async_copy(wd_hbm.at[e], wd_buf.at[s], sems.at[s]).start()


def _wait_fetch(wgu_hbm, wd_hbm, wgu_buf, wd_buf, sems, e, s):
    pltpu.make_async_copy(wgu_hbm.at[e], wgu_buf.at[s], sems.at[s]).wait()
    pltpu.make_async_copy(wd_hbm.at[e], wd_buf.at[s], sems.at[s]).wait()


def _group_body(te_ref, na_ref, first_ref, slot_ref, nexte_ref,
                xs_ref, wgu_hbm, wd_hbm,
                ga_ref, gb_ref, ua_ref, ub_ref, da_ref, db_ref, out_ref,
                wgu_buf, wd_buf, wg_bf, wu_bf, wd_bf, sems):
    i = pl.program_id(0)
    s = slot_ref[i]
    e = te_ref[i]

    @pl.when(i == 0)
    def _():
        _start_fetch(wgu_hbm, wd_hbm, wgu_buf, wd_buf, sems, e, s)

    @pl.when(first_ref[i] == 1)
    def _():
        _wait_fetch(wgu_hbm, wd_hbm, wgu_buf, wd_buf, sems, e, s)
        nxt = nexte_ref[i]

        @pl.when(nxt >= 0)
        def _():
            _start_fetch(wgu_hbm, wd_hbm, wgu_buf, wd_buf, sems,
                         nxt, (s + 1) % 2)

        lg = jax.lax.dot_general(
            ga_ref[0].astype(jnp.bfloat16), gb_ref[0].astype(jnp.bfloat16),
            (((0,), (1,)), ((), ())), preferred_element_type=jnp.float32)
        lu = jax.lax.dot_general(
            ua_ref[0].astype(jnp.bfloat16), ub_ref[0].astype(jnp.bfloat16),
            (((0,), (1,)), ((), ())), preferred_element_type=jnp.float32)
        ld = jax.lax.dot_general(
            da_ref[0].astype(jnp.bfloat16), db_ref[0].astype(jnp.bfloat16),
            (((0,), (1,)), ((), ())), preferred_element_type=jnp.float32)
        wraw = wgu_buf[pl.ds(s, 1)][0]
        wg_bf[pl.ds(s, 1)] = (wraw[:, :_I] + lg).astype(jnp.bfloat16)[None]
        wu_bf[pl.ds(s, 1)] = (wraw[:, _I:] + lu).astype(jnp.bfloat16)[None]
        wd_bf[pl.ds(s, 1)] = (wd_buf[pl.ds(s, 1)][0] + ld
                              ).astype(jnp.bfloat16)[None]

    @pl.when(i < na_ref[0])
    def _():
        xb = xs_ref[...]
        gate = jnp.dot(xb, wg_bf[pl.ds(s, 1)][0],
                       preferred_element_type=jnp.float32)
        up = jnp.dot(xb, wu_bf[pl.ds(s, 1)][0],
                     preferred_element_type=jnp.float32)
        act = (gate * jax.nn.sigmoid(gate) * up).astype(jnp.bfloat16)
        out_ref[...] = jnp.dot(act, wd_bf[pl.ds(s, 1)][0],
                               preferred_element_type=jnp.float32)


def _combine_body(flat_ref, tw_ref, out_ref):
    f = flat_ref[...]
    w = tw_ref[...]
    out_ref[...] = f[:, :_H] * w[:, 0:1] + f[:, _H:] * w[:, 1:2]


def kernel(hidden_states, topk_ids, topk_weights, gate_a, gate_b, up_a, up_b,
           down_a, down_b, weight_indices, seq_lens, lora_ranks, scalings,
           base_gate_up_weight, base_down_weight):
    adapter = weight_indices[0]
    scaling = scalings[adapter].astype(jnp.float32)
    ga = gate_a[adapter]
    gb = gate_b[adapter] * scaling
    ua = up_a[adapter]
    ub = up_b[adapter] * scaling
    da = down_a[adapter]
    db = down_b[adapter] * scaling

    ids_f = topk_ids.reshape(-1).astype(jnp.int32)
    ohf = (ids_f[:, None] == jnp.arange(_E, dtype=jnp.int32)[None, :]
           ).astype(jnp.float32)
    nb = _T * _K // 128
    blocks = ohf.reshape(nb, 128, _E)
    l128 = jnp.tril(jnp.ones((128, 128), jnp.float32))
    inc = jnp.einsum("ab,nbg->nag", l128, blocks)
    bsum = inc[:, -1, :]
    boff = jnp.cumsum(bsum, axis=0) - bsum
    ranks_f = jnp.sum((inc + boff[:, None, :]).reshape(_T * _K, _E) * ohf,
                      axis=1) - 1.0
    ranks = ranks_f.astype(jnp.int32)
    counts = (boff[-1] + bsum[-1]).astype(jnp.int32)
    tiles_e = (counts + _TM - 1) // _TM
    starts_tile = jnp.concatenate(
        [jnp.zeros((1,), jnp.int32), jnp.cumsum(tiles_e)[:-1].astype(jnp.int32)])
    n_active = jnp.sum(tiles_e).astype(jnp.int32)
    pos = (ohf @ (starts_tile * _TM).astype(jnp.float32)
           ).astype(jnp.int32) + ranks
    tile_ids = jnp.arange(_NT, dtype=jnp.int32)
    te_raw = jnp.sum((tile_ids[:, None] >= starts_tile[None, :]).astype(jnp.int32),
                     axis=1) - 1
    te_last = jnp.sum(
        jnp.where(tile_ids == n_active - 1, te_raw, 0)).astype(jnp.int32)
    tile_expert = jnp.where(tile_ids < n_active, te_raw, te_last).astype(jnp.int32)
    pos2 = pos.reshape(_T, _K)
    pos_e = pos2[:, 0]
    pos_o = pos2[:, 1]

    change = jnp.concatenate(
        [jnp.ones((1,), jnp.int32),
         (tile_expert[1:] != tile_expert[:-1]).astype(jnp.int32)])
    eord = jnp.cumsum(change) - 1
    slot = (eord % 2).astype(jnp.int32)
    jj = jnp.arange(_NT, dtype=jnp.int32)
    mask = (jj[None, :] > jj[:, None]) & (change[None, :] > 0)
    nbi = jnp.min(jnp.where(mask, jj[None, :], _NT), axis=1)
    te_ext = jnp.concatenate([tile_expert, jnp.full((1,), -1, jnp.int32)])
    oh2 = (nbi[:, None] == jnp.arange(_NT + 1, dtype=jnp.int32)[None, :]
           ).astype(jnp.int32)
    next_e = jnp.sum(oh2 * te_ext[None, :], axis=1).astype(jnp.int32)

    x32 = lax.bitcast_convert_type(
        hidden_states.astype(jnp.bfloat16).reshape(_T, _H // 2, 2), jnp.int32)
    xs32 = _sc_sort()(x32, pos_e, pos_o)
    xs = lax.bitcast_convert_type(xs32, jnp.bfloat16).reshape(_NP, _H)

    idx_e = lambda i, te, na, fi, sl, ne: (te[i], 0, 0)
    rows_out = pl.pallas_call(
        _group_body,
        grid_spec=pltpu.PrefetchScalarGridSpec(
            num_scalar_prefetch=5,
            grid=(_NT,),
            in_specs=[
                pl.BlockSpec((_TM, _H), lambda i, te, na, fi, sl, ne: (i, 0)),
                pl.BlockSpec(memory_space=pltpu.HBM),
                pl.BlockSpec(memory_space=pltpu.HBM),
                pl.BlockSpec((1, _R, _H), idx_e),
                pl.BlockSpec((1, _I, _R), idx_e),
                pl.BlockSpec((1, _R, _H), idx_e),
                pl.BlockSpec((1, _I, _R), idx_e),
                pl.BlockSpec((1, _R, _I), idx_e),
                pl.BlockSpec((1, _H, _R), idx_e),
            ],
            out_specs=pl.BlockSpec((_TM, _H), lambda i, te, na, fi, sl, ne: (i, 0)),
            scratch_shapes=[
                pltpu.VMEM((2, _H, 2 * _I), jnp.float32),
                pltpu.VMEM((2, _I, _H), jnp.float32),
                pltpu.VMEM((2, _H, _I), jnp.bfloat16),
                pltpu.VMEM((2, _H, _I), jnp.bfloat16),
                pltpu.VMEM((2, _I, _H), jnp.bfloat16),
                pltpu.SemaphoreType.DMA((2,)),
            ],
        ),
        out_shape=jax.ShapeDtypeStruct((_NP, _H), jnp.float32),
    )(tile_expert, n_active.reshape(1), change, slot, next_e, xs,
      base_gate_up_weight, base_down_weight, ga, gb, ua, ub, da, db)

    flat = _sc_unsort()(rows_out, pos)

    out = pl.pallas_call(
        _combine_body,
        grid=(1,),
        in_specs=[
            pl.BlockSpec((_T, _K * _H), lambda i: (0, 0)),
            pl.BlockSpec((_T, _K), lambda i: (0, 0)),
        ],
        out_specs=pl.BlockSpec((_T, _H), lambda i: (0, 0)),
        out_shape=jax.ShapeDtypeStruct((_T, _H), jnp.float32),
    )(flat.reshape(_T, _K * _H), topk_weights.astype(jnp.float32))
    return out


# speedup vs baseline: 1.6501x; 1.6501x over previous
import functools

import jax
import jax.numpy as jnp
from jax import lax
from jax.experimental import pallas as pl
from jax.experimental.pallas import tpu as pltpu
from jax.experimental.pallas import tpu_sc as plsc

_T, _H, _I, _E, _R, _K = 2048, 768, 1536, 8, 16, 2
_TM = 512
_NT = _T * _K // _TM + _E - 1
_NP = _NT * _TM
_NW = 32
_RPW = _T * _K // _NW


def _dotT(a, b):
    return jax.lax.dot_general(
        a, b, (((1,), (1,)), ((), ())), preferred_element_type=jnp.float32)


_TPW = _T // _NW


def _sc_sort_body(x_hbm, pose_hbm, poso_hbm, xs_hbm, pe_v, po_v, buf_v,
                  sem_e, sem_o):
    wid = lax.axis_index("s") * 2 + lax.axis_index("c")
    base = wid * _TPW
    pltpu.sync_copy(pose_hbm.at[pl.ds(base, _TPW)], pe_v)
    pltpu.sync_copy(poso_hbm.at[pl.ds(base, _TPW)], po_v)
    pltpu.sync_copy(x_hbm.at[pl.ds(base, _TPW)], buf_v)
    ce = pltpu.make_async_copy(buf_v, xs_hbm.at[pe_v], sem_e)
    co = pltpu.make_async_copy(buf_v, xs_hbm.at[po_v], sem_o)
    ce.start()
    co.start()
    ce.wait()
    co.wait()


def _sc_sort():
    return pl.kernel(
        _sc_sort_body,
        out_type=jax.ShapeDtypeStruct((_NP, _H), jnp.float32),
        mesh=plsc.VectorSubcoreMesh(core_axis_name="c", subcore_axis_name="s"),
        scratch_types=[
            pltpu.VMEM((_TPW,), jnp.int32),
            pltpu.VMEM((_TPW,), jnp.int32),
            pltpu.VMEM((_TPW, _H), jnp.float32),
            pltpu.SemaphoreType.DMA,
            pltpu.SemaphoreType.DMA,
        ],
    )


def _sc_unsort_body(rows_hbm, pos_hbm, flat_hbm, pos_v, buf_v, sem_g):
    wid = lax.axis_index("s") * 2 + lax.axis_index("c")
    base = wid * _RPW
    pltpu.sync_copy(pos_hbm.at[pl.ds(base, _RPW)], pos_v)
    pltpu.async_copy(rows_hbm.at[pos_v], buf_v, sem_g).wait()
    pltpu.sync_copy(buf_v, flat_hbm.at[pl.ds(base, _RPW)])


def _sc_unsort():
    return pl.kernel(
        _sc_unsort_body,
        out_type=jax.ShapeDtypeStruct((_T * _K, _H), jnp.float32),
        mesh=plsc.VectorSubcoreMesh(core_axis_name="c", subcore_axis_name="s"),
        scratch_types=[
            pltpu.VMEM((_RPW,), jnp.int32),
            pltpu.VMEM((_RPW, _H), jnp.float32),
            pltpu.SemaphoreType.DMA,
        ],
    )


def _start_fetch(wgu_hbm, wd_hbm, wgu_buf, wd_buf, sems, e, s):
    pltpu.make_async_copy(wgu_hbm.at[e], wgu_buf.at[s], sems.at[s]).start()
    pltpu.make_async_copy(wd_hbm.at[e], wd_buf.at[s], sems.at[s]).start()


def _wait_fetch(wgu_hbm, wd_hbm, wgu_buf, wd_buf, sems, e, s):
    pltpu.make_async_copy(wgu_hbm.at[e], wgu_buf.at[s], sems.at[s]).wait()
    pltpu.make_async_copy(wd_hbm.at[e], wd_buf.at[s], sems.at[s]).wait()


def _group_body(te_ref, na_ref, first_ref, slot_ref, nexte_ref,
                xs_ref, wgu_hbm, wd_hbm,
                ga_ref, gb_ref, ua_ref, ub_ref, da_ref, db_ref, out_ref,
                wgu_buf, wd_buf, wg_bf, wu_bf, wd_bf, sems):
    i = pl.program_id(0)
    s = slot_ref[i]
    e = te_ref[i]

    @pl.when(i == 0)
    def _():
        _start_fetch(wgu_hbm, wd_hbm, wgu_buf, wd_buf, sems, e, s)

    @pl.when(first_ref[i] == 1)
    def _():
        _wait_fetch(wgu_hbm, wd_hbm, wgu_buf, wd_buf, sems, e, s)
        nxt = nexte_ref[i]

        @pl.when(nxt >= 0)
        def _():
            _start_fetch(wgu_hbm, wd_hbm, wgu_buf, wd_buf, sems,
                         nxt, (s + 1) % 2)

        lg = jax.lax.dot_general(
            ga_ref[0].astype(jnp.bfloat16), gb_ref[0].astype(jnp.bfloat16),
            (((0,), (1,)), ((), ())), preferred_element_type=jnp.float32)
        lu = jax.lax.dot_general(
            ua_ref[0].astype(jnp.bfloat16), ub_ref[0].astype(jnp.bfloat16),
            (((0,), (1,)), ((), ())), preferred_element_type=jnp.float32)
        ld = jax.lax.dot_general(
            da_ref[0].astype(jnp.bfloat16), db_ref[0].astype(jnp.bfloat16),
            (((0,), (1,)), ((), ())), preferred_element_type=jnp.float32)
        wraw = wgu_buf[pl.ds(s, 1)][0]
        wg_bf[pl.ds(s, 1)] = (wraw[:, :_I] + lg).astype(jnp.bfloat16)[None]
        wu_bf[pl.ds(s, 1)] = (wraw[:, _I:] + lu).astype(jnp.bfloat16)[None]
        wd_bf[pl.ds(s, 1)] = (wd_buf[pl.ds(s, 1)][0] + ld
                              ).astype(jnp.bfloat16)[None]

    @pl.when(i < na_ref[0])
    def _():
        xb = xs_ref[...].astype(jnp.bfloat16)
        gate = jnp.dot(xb, wg_bf[pl.ds(s, 1)][0],
                       preferred_element_type=jnp.float32)
        up = jnp.dot(xb, wu_bf[pl.ds(s, 1)][0],
                     preferred_element_type=jnp.float32)
        act = (gate * jax.nn.sigmoid(gate) * up).astype(jnp.bfloat16)
        out_ref[...] = jnp.dot(act, wd_bf[pl.ds(s, 1)][0],
                               preferred_element_type=jnp.float32)


def _combine_body(flat_ref, tw_ref, out_ref):
    f = flat_ref[...]
    w = tw_ref[...]
    out_ref[...] = f[:, :_H] * w[:, 0:1] + f[:, _H:] * w[:, 1:2]


def kernel(hidden_states, topk_ids, topk_weights, gate_a, gate_b, up_a, up_b,
           down_a, down_b, weight_indices, seq_lens, lora_ranks, scalings,
           base_gate_up_weight, base_down_weight):
    adapter = weight_indices[0]
    scaling = scalings[adapter].astype(jnp.float32)
    ga = gate_a[adapter]
    gb = gate_b[adapter] * scaling
    ua = up_a[adapter]
    ub = up_b[adapter] * scaling
    da = down_a[adapter]
    db = down_b[adapter] * scaling

    ids_f = topk_ids.reshape(-1).astype(jnp.int32)
    ohf = (ids_f[:, None] == jnp.arange(_E, dtype=jnp.int32)[None, :]
           ).astype(jnp.float32)
    nb = _T * _K // 128
    blocks = ohf.reshape(nb, 128, _E)
    l128 = jnp.tril(jnp.ones((128, 128), jnp.float32))
    inc = jnp.einsum("ab,nbg->nag", l128, blocks)
    bsum = inc[:, -1, :]
    boff = jnp.cumsum(bsum, axis=0) - bsum
    ranks_f = jnp.sum((inc + boff[:, None, :]).reshape(_T * _K, _E) * ohf,
                      axis=1) - 1.0
    ranks = ranks_f.astype(jnp.int32)
    counts = (boff[-1] + bsum[-1]).astype(jnp.int32)
    tiles_e = (counts + _TM - 1) // _TM
    starts_tile = jnp.concatenate(
        [jnp.zeros((1,), jnp.int32), jnp.cumsum(tiles_e)[:-1].astype(jnp.int32)])
    n_active = jnp.sum(tiles_e).astype(jnp.int32)
    pos = (ohf @ (starts_tile * _TM).astype(jnp.float32)
           ).astype(jnp.int32) + ranks
    tile_ids = jnp.arange(_NT, dtype=jnp.int32)
    te_raw = jnp.sum((tile_ids[:, None] >= starts_tile[None, :]).astype(jnp.int32),
                     axis=1) - 1
    te_last = jnp.sum(
        jnp.where(tile_ids == n_active - 1, te_raw, 0)).astype(jnp.int32)
    tile_expert = jnp.where(tile_ids < n_active, te_raw, te_last).astype(jnp.int32)
    pos2 = pos.reshape(_T, _K)
    pos_e = pos2[:, 0]
    pos_o = pos2[:, 1]

    change = jnp.concatenate(
        [jnp.ones((1,), jnp.int32),
         (tile_expert[1:] != tile_expert[:-1]).astype(jnp.int32)])
    eord = jnp.cumsum(change) - 1
    slot = (eord % 2).astype(jnp.int32)
    jj = jnp.arange(_NT, dtype=jnp.int32)
    mask = (jj[None, :] > jj[:, None]) & (change[None, :] > 0)
    nbi = jnp.min(jnp.where(mask, jj[None, :], _NT), axis=1)
    te_ext = jnp.concatenate([tile_expert, jnp.full((1,), -1, jnp.int32)])
    oh2 = (nbi[:, None] == jnp.arange(_NT + 1, dtype=jnp.int32)[None, :]
           ).astype(jnp.int32)
    next_e = jnp.sum(oh2 * te_ext[None, :], axis=1).astype(jnp.int32)

    xs = _sc_sort()(hidden_states, pos_e, pos_o)

    idx_e = lambda i, te, na, fi, sl, ne: (te[i], 0, 0)
    rows_out = pl.pallas_call(
        _group_body,
        grid_spec=pltpu.PrefetchScalarGridSpec(
            num_scalar_prefetch=5,
            grid=(_NT,),
            in_specs=[
                pl.BlockSpec((_TM, _H), lambda i, te, na, fi, sl, ne: (i, 0)),
                pl.BlockSpec(memory_space=pltpu.HBM),
                pl.BlockSpec(memory_space=pltpu.HBM),
                pl.BlockSpec((1, _R, _H), idx_e),
                pl.BlockSpec((1, _I, _R), idx_e),
                pl.BlockSpec((1, _R, _H), idx_e),
                pl.BlockSpec((1, _I, _R), idx_e),
                pl.BlockSpec((1, _R, _I), idx_e),
                pl.BlockSpec((1, _H, _R), idx_e),
            ],
            out_specs=pl.BlockSpec((_TM, _H), lambda i, te, na, fi, sl, ne: (i, 0)),
            scratch_shapes=[
                pltpu.VMEM((2, _H, 2 * _I), jnp.float32),
                pltpu.VMEM((2, _I, _H), jnp.float32),
                pltpu.VMEM((2, _H, _I), jnp.bfloat16),
                pltpu.VMEM((2, _H, _I), jnp.bfloat16),
                pltpu.VMEM((2, _I, _H), jnp.bfloat16),
                pltpu.SemaphoreType.DMA((2,)),
            ],
        ),
        out_shape=jax.ShapeDtypeStruct((_NP, _H), jnp.float32),
    )(tile_expert, n_active.reshape(1), change, slot, next_e, xs,
      base_gate_up_weight, base_down_weight, ga, gb, ua, ub, da, db)

    flat = _sc_unsort()(rows_out, pos)

    out = pl.pallas_call(
        _combine_body,
        grid=(1,),
        in_specs=[
            pl.BlockSpec((_T, _K * _H), lambda i: (0, 0)),
            pl.BlockSpec((_T, _K), lambda i: (0, 0)),
        ],
        out_specs=pl.BlockSpec((_T, _H), lambda i: (0, 0)),
        out_shape=jax.ShapeDtypeStruct((_T, _H), jnp.float32),
    )(flat.reshape(_T, _K * _H), topk_weights.astype(jnp.float32))
    return out


# speedup vs baseline: 1.7375x; 1.0530x over previous
import jax
import jax.numpy as jnp
from jax import lax
from jax.experimental import pallas as pl
from jax.experimental.pallas import tpu as pltpu
from jax.experimental.pallas import tpu_sc as plsc

_T, _H, _I, _E, _R, _K = 2048, 768, 1536, 8, 16, 2
_TM = 256
_NT = _T * _K // _TM + _E - 1
_NP = _NT * _TM
_NW = 32
_RPW = _T * _K // _NW


_TPW = _T // _NW


def _sc_sort_body(x_hbm, pose_hbm, poso_hbm, xs_hbm, pe_v, po_v, buf_v,
                  sem_e, sem_o):
    wid = lax.axis_index("s") * 2 + lax.axis_index("c")
    base = wid * _TPW
    pltpu.sync_copy(pose_hbm.at[pl.ds(base, _TPW)], pe_v)
    pltpu.sync_copy(poso_hbm.at[pl.ds(base, _TPW)], po_v)
    pltpu.sync_copy(x_hbm.at[pl.ds(base, _TPW)], buf_v)
    ce = pltpu.make_async_copy(buf_v, xs_hbm.at[pe_v], sem_e)
    co = pltpu.make_async_copy(buf_v, xs_hbm.at[po_v], sem_o)
    ce.start()
    co.start()
    ce.wait()
    co.wait()


def _sc_sort():
    return pl.kernel(
        _sc_sort_body,
        out_type=jax.ShapeDtypeStruct((_NP, _H), jnp.float32),
        mesh=plsc.VectorSubcoreMesh(core_axis_name="c", subcore_axis_name="s"),
        scratch_types=[
            pltpu.VMEM((_TPW,), jnp.int32),
            pltpu.VMEM((_TPW,), jnp.int32),
            pltpu.VMEM((_TPW, _H), jnp.float32),
            pltpu.SemaphoreType.DMA,
            pltpu.SemaphoreType.DMA,
        ],
    )


def _sc_unsort_body(rows_hbm, pos_hbm, flat_hbm, pos_v, buf_v, sem_g):
    wid = lax.axis_index("s") * 2 + lax.axis_index("c")
    base = wid * _RPW
    pltpu.sync_copy(pos_hbm.at[pl.ds(base, _RPW)], pos_v)
    pltpu.async_copy(rows_hbm.at[pos_v], buf_v, sem_g).wait()
    pltpu.sync_copy(buf_v, flat_hbm.at[pl.ds(base, _RPW)])


def _sc_unsort():
    return pl.kernel(
        _sc_unsort_body,
        out_type=jax.ShapeDtypeStruct((_T * _K, _H), jnp.float32),
        mesh=plsc.VectorSubcoreMesh(core_axis_name="c", subcore_axis_name="s"),
        scratch_types=[
            pltpu.VMEM((_RPW,), jnp.int32),
            pltpu.VMEM((_RPW, _H), jnp.float32),
            pltpu.SemaphoreType.DMA,
        ],
    )


def _start_fetch(wgu_hbm, wd_hbm, wgu_buf, wd_buf, sems, e, s):
    pltpu.make_async_copy(wgu_hbm.at[e], wgu_buf.at[s], sems.at[s]).start()
    pltpu.make_async_copy(wd_hbm.at[e], wd_buf.at[s], sems.at[s]).start()


def _wait_fetch(wgu_hbm, wd_hbm, wgu_buf, wd_buf, sems, e, s):
    pltpu.make_async_copy(wgu_hbm.at[e], wgu_buf.at[s], sems.at[s]).wait()
    pltpu.make_async_copy(wd_hbm.at[e], wd_buf.at[s], sems.at[s]).wait()


def _group_body(te_ref, na_ref, first_ref, slot_ref, nexte_ref,
                xs_ref, wgu_hbm, wd_hbm,
                ga_ref, gb_ref, ua_ref, ub_ref, da_ref, db_ref, out_ref,
                wgu_buf, wd_buf, wg_bf, wu_bf, wd_bf, sems):
    i = pl.program_id(0)
    s = slot_ref[i]
    e = te_ref[i]

    @pl.when(i == 0)
    def _():
        _start_fetch(wgu_hbm, wd_hbm, wgu_buf, wd_buf, sems, e, s)

    @pl.when(first_ref[i] == 1)
    def _():
        _wait_fetch(wgu_hbm, wd_hbm, wgu_buf, wd_buf, sems, e, s)
        nxt = nexte_ref[i]

        @pl.when(nxt >= 0)
        def _():
            _start_fetch(wgu_hbm, wd_hbm, wgu_buf, wd_buf, sems,
                         nxt, (s + 1) % 2)

        lg = jax.lax.dot_general(
            ga_ref[0].astype(jnp.bfloat16), gb_ref[0].astype(jnp.bfloat16),
            (((0,), (1,)), ((), ())), preferred_element_type=jnp.float32)
        lu = jax.lax.dot_general(
            ua_ref[0].astype(jnp.bfloat16), ub_ref[0].astype(jnp.bfloat16),
            (((0,), (1,)), ((), ())), preferred_element_type=jnp.float32)
        ld = jax.lax.dot_general(
            da_ref[0].astype(jnp.bfloat16), db_ref[0].astype(jnp.bfloat16),
            (((0,), (1,)), ((), ())), preferred_element_type=jnp.float32)
        wraw = wgu_buf[pl.ds(s, 1)][0]
        wg_bf[pl.ds(s, 1)] = (wraw[:, :_I] + lg).astype(jnp.bfloat16)[None]
        wu_bf[pl.ds(s, 1)] = (wraw[:, _I:] + lu).astype(jnp.bfloat16)[None]
        wd_bf[pl.ds(s, 1)] = (wd_buf[pl.ds(s, 1)][0] + ld
                              ).astype(jnp.bfloat16)[None]

    @pl.when(i < na_ref[0])
    def _():
        xb = xs_ref[...].astype(jnp.bfloat16)
        gate = jnp.dot(xb, wg_bf[pl.ds(s, 1)][0],
                       preferred_element_type=jnp.float32)
        up = jnp.dot(xb, wu_bf[pl.ds(s, 1)][0],
                     preferred_element_type=jnp.float32)
        act = (gate * jax.nn.sigmoid(gate) * up).astype(jnp.bfloat16)
        out_ref[...] = jnp.dot(act, wd_bf[pl.ds(s, 1)][0],
                               preferred_element_type=jnp.float32)


def _combine_body(flat_ref, tw_ref, out_ref):
    f = flat_ref[...]
    w = tw_ref[...]
    out_ref[...] = f[:, :_H] * w[:, 0:1] + f[:, _H:] * w[:, 1:2]


def kernel(hidden_states, topk_ids, topk_weights, gate_a, gate_b, up_a, up_b,
           down_a, down_b, weight_indices, seq_lens, lora_ranks, scalings,
           base_gate_up_weight, base_down_weight):
    adapter = weight_indices[0]
    scaling = scalings[adapter].astype(jnp.float32)
    ga = gate_a[adapter]
    gb = gate_b[adapter] * scaling
    ua = up_a[adapter]
    ub = up_b[adapter] * scaling
    da = down_a[adapter]
    db = down_b[adapter] * scaling

    ids_f = topk_ids.reshape(-1).astype(jnp.int32)
    ohf = (ids_f[:, None] == jnp.arange(_E, dtype=jnp.int32)[None, :]
           ).astype(jnp.float32)
    nb = _T * _K // 128
    blocks = ohf.reshape(nb, 128, _E)
    l128 = jnp.tril(jnp.ones((128, 128), jnp.float32))
    inc = jnp.einsum("ab,nbg->nag", l128, blocks)
    bsum = inc[:, -1, :]
    boff = jnp.cumsum(bsum, axis=0) - bsum
    ranks_f = jnp.sum((inc + boff[:, None, :]).reshape(_T * _K, _E) * ohf,
                      axis=1) - 1.0
    ranks = ranks_f.astype(jnp.int32)
    counts = (boff[-1] + bsum[-1]).astype(jnp.int32)
    tiles_e = (counts + _TM - 1) // _TM
    starts_tile = jnp.concatenate(
        [jnp.zeros((1,), jnp.int32), jnp.cumsum(tiles_e)[:-1].astype(jnp.int32)])
    n_active = jnp.sum(tiles_e).astype(jnp.int32)
    pos = (ohf @ (starts_tile * _TM).astype(jnp.float32)
           ).astype(jnp.int32) + ranks
    tile_ids = jnp.arange(_NT, dtype=jnp.int32)
    te_raw = jnp.sum((tile_ids[:, None] >= starts_tile[None, :]).astype(jnp.int32),
                     axis=1) - 1
    te_last = jnp.sum(
        jnp.where(tile_ids == n_active - 1, te_raw, 0)).astype(jnp.int32)
    tile_expert = jnp.where(tile_ids < n_active, te_raw, te_last).astype(jnp.int32)
    pos2 = pos.reshape(_T, _K)
    pos_e = pos2[:, 0]
    pos_o = pos2[:, 1]

    change = jnp.concatenate(
        [jnp.ones((1,), jnp.int32),
         (tile_expert[1:] != tile_expert[:-1]).astype(jnp.int32)])
    eord = jnp.cumsum(change) - 1
    slot = (eord % 2).astype(jnp.int32)
    jj = jnp.arange(_NT, dtype=jnp.int32)
    mask = (jj[None, :] > jj[:, None]) & (change[None, :] > 0)
    nbi = jnp.min(jnp.where(mask, jj[None, :], _NT), axis=1)
    te_ext = jnp.concatenate([tile_expert, jnp.full((1,), -1, jnp.int32)])
    oh2 = (nbi[:, None] == jnp.arange(_NT + 1, dtype=jnp.int32)[None, :]
           ).astype(jnp.int32)
    next_e = jnp.sum(oh2 * te_ext[None, :], axis=1).astype(jnp.int32)

    xs = _sc_sort()(hidden_states, pos_e, pos_o)

    idx_e = lambda i, te, na, fi, sl, ne: (te[i], 0, 0)
    rows_out = pl.pallas_call(
        _group_body,
        grid_spec=pltpu.PrefetchScalarGridSpec(
            num_scalar_prefetch=5,
            grid=(_NT,),
            in_specs=[
                pl.BlockSpec((_TM, _H), lambda i, te, na, fi, sl, ne: (i, 0)),
                pl.BlockSpec(memory_space=pltpu.HBM),
                pl.BlockSpec(memory_space=pltpu.HBM),
                pl.BlockSpec((1, _R, _H), idx_e),
                pl.BlockSpec((1, _I, _R), idx_e),
                pl.BlockSpec((1, _R, _H), idx_e),
                pl.BlockSpec((1, _I, _R), idx_e),
                pl.BlockSpec((1, _R, _I), idx_e),
                pl.BlockSpec((1, _H, _R), idx_e),
            ],
            out_specs=pl.BlockSpec((_TM, _H), lambda i, te, na, fi, sl, ne: (i, 0)),
            scratch_shapes=[
                pltpu.VMEM((2, _H, 2 * _I), jnp.float32),
                pltpu.VMEM((2, _I, _H), jnp.float32),
                pltpu.VMEM((2, _H, _I), jnp.bfloat16),
                pltpu.VMEM((2, _H, _I), jnp.bfloat16),
                pltpu.VMEM((2, _I, _H), jnp.bfloat16),
                pltpu.SemaphoreType.DMA((2,)),
            ],
        ),
        out_shape=jax.ShapeDtypeStruct((_NP, _H), jnp.float32),
    )(tile_expert, n_active.reshape(1), change, slot, next_e, xs,
      base_gate_up_weight, base_down_weight, ga, gb, ua, ub, da, db)

    flat = _sc_unsort()(rows_out, pos)

    out = pl.pallas_call(
        _combine_body,
        grid=(1,),
        in_specs=[
            pl.BlockSpec((_T, _K * _H), lambda i: (0, 0)),
            pl.BlockSpec((_T, _K), lambda i: (0, 0)),
        ],
        out_specs=pl.BlockSpec((_T, _H), lambda i: (0, 0)),
        out_shape=jax.ShapeDtypeStruct((_T, _H), jnp.float32),
    )(flat.reshape(_T, _K * _H), topk_weights.astype(jnp.float32))
    return out
